# Initial kernel scaffold; baseline (speedup 1.0000x reference)
import jax
import jax.numpy as jnp
from jax.experimental import pallas as pl


def kernel(x, edge_index, Wl0, bl0, Wr0, bn_g0, bn_b0, Wl1, bl1, Wr1, bn_g1, bn_b1, Wl2, bl2, Wr2, bn_g2, bn_b2, Wl3, bl3, Wr3, bn_g3, bn_b3, Wc, bc):
    raise NotImplementedError("write your pallas kernel here")


# speedup vs baseline: 4.0730x; 4.0730x over previous
import functools

import jax
import jax.numpy as jnp
from jax import lax
from jax.experimental import pallas as pl
from jax.experimental.pallas import tpu as pltpu
from jax.experimental.pallas import tpu_sc as plsc

N = 10000
D = 128
E = 320000
EPS = 1e-5

NW = 32
LANES = 128
NBLK = -(-E // (NW * LANES))
EP = NW * NBLK * LANES
NP = 10240
ZROWS = 64
RPT = NP // 16

_mesh = plsc.VectorSubcoreMesh(core_axis_name="c", subcore_axis_name="s")


def _zero_vmem_2d(ref, nrows, ncols):
    def row(i, _):
        def col(j, _):
            ref[i, pl.ds(j * 16, 16)] = jnp.zeros((16,), jnp.float32)
            return 0
        return lax.fori_loop(0, ncols // 16, col, 0)
    lax.fori_loop(0, nrows, row, 0)


def _sc_agg_body(h_hbm, src_hbm, dst_hbm, out_hbm, sidx, didx, rows, zbuf, aggS, sem):
    c = lax.axis_index("c")
    s = lax.axis_index("s")
    wid = s * 2 + c

    _zero_vmem_2d(zbuf, ZROWS, D)
    base = s * RPT
    for k in range(RPT // ZROWS):
        pltpu.sync_copy(zbuf, aggS.at[pl.ds(base + k * ZROWS, ZROWS)])
    plsc.subcore_barrier()

    pltpu.sync_copy(src_hbm.at[wid], sidx)
    pltpu.sync_copy(dst_hbm.at[wid], didx)

    def step(j, _):
        pltpu.async_copy(h_hbm.at[sidx.at[j]], rows, sem).wait()
        pltpu.sync_copy(rows, aggS.at[didx.at[j]], add=True)
        return 0
    lax.fori_loop(0, NBLK, step, 0)

    plsc.subcore_barrier()
    pltpu.sync_copy(aggS.at[pl.ds(base, RPT)], out_hbm.at[c, pl.ds(base, RPT)])


_sc_agg = functools.partial(
    pl.kernel,
    out_type=jax.ShapeDtypeStruct((2, NP, D), jnp.float32),
    mesh=_mesh,
    scratch_types=[
        pltpu.VMEM((NBLK, LANES), jnp.int32),
        pltpu.VMEM((NBLK, LANES), jnp.int32),
        pltpu.VMEM((LANES, D), jnp.float32),
        pltpu.VMEM((ZROWS, D), jnp.float32),
        pltpu.VMEM_SHARED((NP, D), jnp.float32),
        pltpu.SemaphoreType.DMA,
    ],
)(_sc_agg_body)


def _sc_deg_body(dst_hbm, out_hbm, didx, ones, zbuf, degS):
    c = lax.axis_index("c")
    s = lax.axis_index("s")
    wid = s * 2 + c

    _zero_vmem_2d(zbuf, ZROWS, D)
    base = s * RPT
    for k in range(RPT // ZROWS):
        pltpu.sync_copy(zbuf, degS.at[pl.ds(base + k * ZROWS, ZROWS)])

    def orow(i, _):
        def ocol(j, _):
            ones[i, pl.ds(j * 16, 16)] = jnp.ones((16,), jnp.float32)
            return 0
        return lax.fori_loop(0, D // 16, ocol, 0)
    lax.fori_loop(0, LANES, orow, 0)
    plsc.subcore_barrier()

    pltpu.sync_copy(dst_hbm.at[wid], didx)

    def step(j, _):
        pltpu.sync_copy(ones, degS.at[didx.at[j]], add=True)
        return 0
    lax.fori_loop(0, NBLK, step, 0)

    plsc.subcore_barrier()
    pltpu.sync_copy(degS.at[pl.ds(base, RPT)], out_hbm.at[c, pl.ds(base, RPT)])


_sc_deg = functools.partial(
    pl.kernel,
    out_type=jax.ShapeDtypeStruct((2, NP, D), jnp.float32),
    mesh=_mesh,
    scratch_types=[
        pltpu.VMEM((NBLK, LANES), jnp.int32),
        pltpu.VMEM((LANES, D), jnp.float32),
        pltpu.VMEM((ZROWS, D), jnp.float32),
        pltpu.VMEM_SHARED((NP, D), jnp.float32),
    ],
)(_sc_deg_body)


ROWB = 2000


def _tc_layer_body(agg, deg, h, Wl, Wr, sc, be, out):
    a = agg[0] + agg[1]
    dg = deg[0, :, :1] + deg[1, :, :1]
    am = a * (1.0 / jnp.maximum(dg, 1.0))
    acc = jnp.dot(am, Wl[...], preferred_element_type=jnp.float32)
    acc = acc + jnp.dot(h[...], Wr[...], preferred_element_type=jnp.float32)
    out[...] = jnp.maximum(acc * sc[...] + be[...], 0.0)


def _tc_last_body(agg, deg, h, Wl, Wr, sc, be, Wc, bc, out):
    a = agg[0] + agg[1]
    dg = deg[0, :, :1] + deg[1, :, :1]
    am = a * (1.0 / jnp.maximum(dg, 1.0))
    acc = jnp.dot(am, Wl[...], preferred_element_type=jnp.float32)
    acc = acc + jnp.dot(h[...], Wr[...], preferred_element_type=jnp.float32)
    hr = jnp.maximum(acc * sc[...] + be[...], 0.0)
    out[...] = jnp.dot(hr, Wc[...], preferred_element_type=jnp.float32) + bc[...]


def _tc_layer(agg, deg, h, Wl, Wr, sc, be):
    grid = (N // ROWB,)
    return pl.pallas_call(
        _tc_layer_body,
        grid=grid,
        in_specs=[
            pl.BlockSpec((2, ROWB, D), lambda i: (0, i, 0)),
            pl.BlockSpec((2, ROWB, D), lambda i: (0, i, 0)),
            pl.BlockSpec((ROWB, D), lambda i: (i, 0)),
            pl.BlockSpec((D, D), lambda i: (0, 0)),
            pl.BlockSpec((D, D), lambda i: (0, 0)),
            pl.BlockSpec((1, D), lambda i: (0, 0)),
            pl.BlockSpec((1, D), lambda i: (0, 0)),
        ],
        out_specs=pl.BlockSpec((ROWB, D), lambda i: (i, 0)),
        out_shape=jax.ShapeDtypeStruct((N, D), jnp.float32),
    )(agg, deg, h, Wl, Wr, sc, be)


def _tc_last(agg, deg, h, Wl, Wr, sc, be, Wc, bc):
    grid = (N // ROWB,)
    return pl.pallas_call(
        _tc_last_body,
        grid=grid,
        in_specs=[
            pl.BlockSpec((2, ROWB, D), lambda i: (0, i, 0)),
            pl.BlockSpec((2, ROWB, D), lambda i: (0, i, 0)),
            pl.BlockSpec((ROWB, D), lambda i: (i, 0)),
            pl.BlockSpec((D, D), lambda i: (0, 0)),
            pl.BlockSpec((D, D), lambda i: (0, 0)),
            pl.BlockSpec((1, D), lambda i: (0, 0)),
            pl.BlockSpec((1, D), lambda i: (0, 0)),
            pl.BlockSpec((D, 2), lambda i: (0, 0)),
            pl.BlockSpec((1, 2), lambda i: (0, 0)),
        ],
        out_specs=pl.BlockSpec((ROWB, 2), lambda i: (i, 0)),
        out_shape=jax.ShapeDtypeStruct((N, 2), jnp.float32),
    )(agg, deg, h, Wl, Wr, sc, be, Wc, bc)


def kernel(x, edge_index, Wl0, bl0, Wr0, bn_g0, bn_b0, Wl1, bl1, Wr1, bn_g1, bn_b1,
           Wl2, bl2, Wr2, bn_g2, bn_b2, Wl3, bl3, Wr3, bn_g3, bn_b3, Wc, bc):
    src = edge_index[0].astype(jnp.int32)
    dst = edge_index[1].astype(jnp.int32)
    pad = EP - E
    src_r = jnp.concatenate([src, jnp.zeros((pad,), jnp.int32)]).reshape(NW, NBLK, LANES)
    dst_r = jnp.concatenate([dst, jnp.full((pad,), N, jnp.int32)]).reshape(NW, NBLK, LANES)

    deg = _sc_deg(dst_r)

    scale = 1.0 / jnp.sqrt(1.0 + EPS)
    layers = [
        (Wl0, bl0, Wr0, bn_g0, bn_b0),
        (Wl1, bl1, Wr1, bn_g1, bn_b1),
        (Wl2, bl2, Wr2, bn_g2, bn_b2),
        (Wl3, bl3, Wr3, bn_g3, bn_b3),
    ]

    h = x
    for li, (Wl, bl, Wr, g, b) in enumerate(layers):
        s = (g * scale).reshape(1, D)
        be = (bl * g * scale + b).reshape(1, D)
        agg = _sc_agg(h, src_r, dst_r)
        if li < 3:
            h = _tc_layer(agg, deg, h, Wl, Wr, s, be)
        else:
            out = _tc_last(agg, deg, h, Wl, Wr, s, be, Wc, bc.reshape(1, 2))
    return out
